# initial kernel scaffold (unmeasured)
import jax
import jax.numpy as jnp
from jax import lax
from jax.experimental import pallas as pl
from jax.experimental.pallas import tpu as pltpu

N_DEV = 16
M = 4096
N = 8192
MC = M // N_DEV
B_BF16 = 10
MESH = pltpu.DeviceIdType.MESH


def kernel(x, w_mat):
    p = jnp.dot(
        x.astype(jnp.bfloat16),
        w_mat.astype(jnp.bfloat16),
        preferred_element_type=jnp.float32,
    )

    def body(
        p_ref,
        out_ref,
        pchunk,
        wire_bf,
        wire_f32,
        agbuf,
        stage,
        amax_buf,
        load_sem,
        store_sem,
        rs_send_sems,
        rs_recv_sems,
        ag_send_sems,
        ag_recv_sems,
        am_send_sems,
        am_recv_sems,
        rs_credit,
        ag_credit,
    ):
        d = lax.axis_index("i")
        left = (d - 1) % N_DEV
        right = (d + 1) % N_DEV

        barrier = pltpu.get_barrier_semaphore()
        for nbr in (left, right):
            pltpu.semaphore_signal(barrier, 1, device_id=(nbr,), device_id_type=MESH)
        pltpu.semaphore_wait(barrier, 2)

        def load_p(chunk_idx):
            cp = pltpu.make_async_copy(
                p_ref.at[pl.ds(chunk_idx * MC, MC), :], pchunk, load_sem
            )
            cp.start()
            return cp

        load_p(d).wait()
        wire_bf[0] = pchunk[:, :].astype(jnp.bfloat16)

        acc = None
        for h in range(N_DEV - 1):
            s = h % 2
            r = (h + 1) % 2
            send_bf = h < B_BF16
            recv_bf = send_bf
            src = wire_bf.at[s] if send_bf else wire_f32.at[s]
            dst = wire_bf.at[r] if recv_bf else wire_f32.at[r]
            if h >= 2:
                pltpu.semaphore_wait(rs_credit, 1)
            rdma = pltpu.make_async_remote_copy(
                src_ref=src,
                dst_ref=dst,
                send_sem=rs_send_sems.at[s],
                recv_sem=rs_recv_sems.at[r],
                device_id=(right,),
                device_id_type=MESH,
            )
            rdma.start()
            cp = load_p((d - h - 1) % N_DEV)
            rdma.wait()
            cp.wait()
            recv = wire_bf[r].astype(jnp.float32) if recv_bf else wire_f32[r]
            acc = recv + pchunk[:, :]
            if h < N_DEV - 2:
                if (h + 1) < B_BF16:
                    wire_bf[r] = acc.astype(jnp.bfloat16)
                else:
                    wire_f32[r] = acc
            if 1 <= h <= 13:
                pltpu.semaphore_signal(rs_credit, 1, device_id=(left,), device_id_type=MESH)

        local_amax = jnp.max(jnp.abs(acc))
        amax_buf[pl.ds(d, 1), :] = jnp.full((1, 128), local_amax, jnp.float32)
        am_rdmas = []
        for j in range(1, N_DEV):
            tgt = (d + j) % N_DEV
            rd = pltpu.make_async_remote_copy(
                src_ref=amax_buf.at[pl.ds(d, 1), :],
                dst_ref=amax_buf.at[pl.ds(d, 1), :],
                send_sem=am_send_sems.at[j],
                recv_sem=am_recv_sems.at[j],
                device_id=(tgt,),
                device_id_type=MESH,
            )
            rd.start()
            am_rdmas.append(rd)
        for j in range(1, N_DEV):
            src_dev = (d - j) % N_DEV
            pltpu.make_async_remote_copy(
                src_ref=amax_buf.at[pl.ds(src_dev, 1), :],
                dst_ref=amax_buf.at[pl.ds(src_dev, 1), :],
                send_sem=am_send_sems.at[j],
                recv_sem=am_recv_sems.at[j],
                device_id=(src_dev,),
                device_id_type=MESH,
            ).wait_recv()
        for rd in am_rdmas:
            rd.wait_send()
        amax = jnp.max(amax_buf[:, :])
        scale = amax / 448.0
        inv_scale = 448.0 / amax

        agbuf[0] = (acc * inv_scale).astype(jnp.float8_e4m3fn)

        def put_out(chunk_idx, q_slot):
            stage[:, :] = (
                agbuf[q_slot].astype(jnp.float32) * scale
            ).astype(jnp.bfloat16)
            cp = pltpu.make_async_copy(
                stage, out_ref.at[pl.ds(chunk_idx * MC, MC), :], store_sem
            )
            cp.start()
            cp.wait()

        put_out((d + 1) % N_DEV, 0)

        for g in range(N_DEV - 1):
            s = g % 2
            r = (g + 1) % 2
            if g >= 2:
                pltpu.semaphore_wait(ag_credit, 1)
            rdma = pltpu.make_async_remote_copy(
                src_ref=agbuf.at[s],
                dst_ref=agbuf.at[r],
                send_sem=ag_send_sems.at[s],
                recv_sem=ag_recv_sems.at[r],
                device_id=(right,),
                device_id_type=MESH,
            )
            rdma.start()
            rdma.wait()
            put_out((d - g) % N_DEV, r)
            if 1 <= g <= 13:
                pltpu.semaphore_signal(ag_credit, 1, device_id=(left,), device_id_type=MESH)

    out_shape = jax.ShapeDtypeStruct((M, N), jnp.bfloat16)
    return pl.pallas_call(
        body,
        out_shape=out_shape,
        in_specs=[pl.BlockSpec(memory_space=pltpu.ANY)],
        out_specs=pl.BlockSpec(memory_space=pltpu.ANY),
        scratch_shapes=[
            pltpu.VMEM((MC, N), jnp.float32),
            pltpu.VMEM((2, MC, N), jnp.bfloat16),
            pltpu.VMEM((2, MC, N), jnp.float32),
            pltpu.VMEM((2, MC, N), jnp.float8_e4m3fn),
            pltpu.VMEM((MC, N), jnp.bfloat16),
            pltpu.VMEM((N_DEV, 128), jnp.float32),
            pltpu.SemaphoreType.DMA,
            pltpu.SemaphoreType.DMA,
            pltpu.SemaphoreType.DMA((2,)),
            pltpu.SemaphoreType.DMA((2,)),
            pltpu.SemaphoreType.DMA((2,)),
            pltpu.SemaphoreType.DMA((2,)),
            pltpu.SemaphoreType.DMA((N_DEV,)),
            pltpu.SemaphoreType.DMA((N_DEV,)),
            pltpu.SemaphoreType.REGULAR,
            pltpu.SemaphoreType.REGULAR,
        ],
        compiler_params=pltpu.CompilerParams(collective_id=0),
    )(p)


# baseline (device time: 1538167 ns/iter reference)
import jax
import jax.numpy as jnp
from jax import lax
from jax.experimental import pallas as pl
from jax.experimental.pallas import tpu as pltpu

N_DEV = 16
M = 4096
N = 8192
MC = M // N_DEV
B_BF16 = 10
MESH = pltpu.DeviceIdType.MESH


def kernel(x, w_mat):
    p = jnp.dot(
        x.astype(jnp.bfloat16),
        w_mat.astype(jnp.bfloat16),
        preferred_element_type=jnp.float32,
    )

    def body(
        p_ref,
        out_ref,
        pchunk,
        wire_bf,
        wire_f32,
        agbuf,
        stage,
        amax_buf,
        load_sem,
        store_sem,
        rs_send_sems,
        rs_recv_sems,
        ag_send_sems,
        ag_recv_sems,
        am_send_sems,
        am_recv_sems,
        rs_credit,
        ag_credit,
    ):
        d = lax.axis_index("i")
        left = (d - 1) % N_DEV
        right = (d + 1) % N_DEV

        barrier = pltpu.get_barrier_semaphore()
        for nbr in (left, right):
            pltpu.semaphore_signal(barrier, 1, device_id=(nbr,), device_id_type=MESH)
        pltpu.semaphore_wait(barrier, 2)

        def load_p(chunk_idx):
            cp = pltpu.make_async_copy(
                p_ref.at[pl.ds(chunk_idx * MC, MC), :], pchunk, load_sem
            )
            cp.start()
            return cp

        load_p(d).wait()
        wire_bf[0] = pchunk[:, :].astype(jnp.bfloat16)

        acc = None
        for h in range(N_DEV - 1):
            s = h % 2
            r = (h + 1) % 2
            send_bf = h < B_BF16
            recv_bf = send_bf
            src = wire_bf.at[s] if send_bf else wire_f32.at[s]
            dst = wire_bf.at[r] if recv_bf else wire_f32.at[r]
            if h >= 2:
                pltpu.semaphore_wait(rs_credit, 1)
            rdma = pltpu.make_async_remote_copy(
                src_ref=src,
                dst_ref=dst,
                send_sem=rs_send_sems.at[s],
                recv_sem=rs_recv_sems.at[r],
                device_id=(right,),
                device_id_type=MESH,
            )
            rdma.start()
            cp = load_p((d - h - 1) % N_DEV)
            rdma.wait()
            cp.wait()
            recv = wire_bf[r].astype(jnp.float32) if recv_bf else wire_f32[r]
            acc = recv + pchunk[:, :]
            if h < N_DEV - 2:
                if (h + 1) < B_BF16:
                    wire_bf[r] = acc.astype(jnp.bfloat16)
                else:
                    wire_f32[r] = acc
            if 1 <= h <= 13:
                pltpu.semaphore_signal(rs_credit, 1, device_id=(left,), device_id_type=MESH)

        local_amax = jnp.max(jnp.abs(acc))
        amax_buf[pl.ds(d, 1), :] = jnp.full((1, 128), local_amax, jnp.float32)
        am_rdmas = []
        for j in range(1, N_DEV):
            tgt = (d + j) % N_DEV
            rd = pltpu.make_async_remote_copy(
                src_ref=amax_buf.at[pl.ds(d, 1), :],
                dst_ref=amax_buf.at[pl.ds(d, 1), :],
                send_sem=am_send_sems.at[j],
                recv_sem=am_recv_sems.at[j],
                device_id=(tgt,),
                device_id_type=MESH,
            )
            rd.start()
            am_rdmas.append(rd)
        for j in range(1, N_DEV):
            src_dev = (d - j) % N_DEV
            pltpu.make_async_remote_copy(
                src_ref=amax_buf.at[pl.ds(src_dev, 1), :],
                dst_ref=amax_buf.at[pl.ds(src_dev, 1), :],
                send_sem=am_send_sems.at[j],
                recv_sem=am_recv_sems.at[j],
                device_id=(src_dev,),
                device_id_type=MESH,
            ).wait_recv()
        for rd in am_rdmas:
            rd.wait_send()
        amax = jnp.max(amax_buf[:, :])
        scale = amax / 448.0
        inv_scale = 448.0 / amax

        agbuf[0] = (acc * inv_scale).astype(jnp.float8_e4m3fn)

        def put_out(chunk_idx, q_slot):
            stage[:, :] = (
                agbuf[q_slot].astype(jnp.float32) * scale
            ).astype(jnp.bfloat16)
            cp = pltpu.make_async_copy(
                stage, out_ref.at[pl.ds(chunk_idx * MC, MC), :], store_sem
            )
            cp.start()
            cp.wait()

        put_out((d + 1) % N_DEV, 0)

        for g in range(N_DEV - 1):
            s = g % 2
            r = (g + 1) % 2
            if g >= 2:
                pltpu.semaphore_wait(ag_credit, 1)
            rdma = pltpu.make_async_remote_copy(
                src_ref=agbuf.at[s],
                dst_ref=agbuf.at[r],
                send_sem=ag_send_sems.at[s],
                recv_sem=ag_recv_sems.at[r],
                device_id=(right,),
                device_id_type=MESH,
            )
            rdma.start()
            rdma.wait()
            put_out((d - g) % N_DEV, r)
            if 1 <= g <= 13:
                pltpu.semaphore_signal(ag_credit, 1, device_id=(left,), device_id_type=MESH)

    out_shape = jax.ShapeDtypeStruct((M, N), jnp.bfloat16)
    return pl.pallas_call(
        body,
        out_shape=out_shape,
        in_specs=[pl.BlockSpec(memory_space=pl.ANY)],
        out_specs=pl.BlockSpec(memory_space=pl.ANY),
        scratch_shapes=[
            pltpu.VMEM((MC, N), jnp.float32),
            pltpu.VMEM((2, MC, N), jnp.bfloat16),
            pltpu.VMEM((2, MC, N), jnp.float32),
            pltpu.VMEM((2, MC, N), jnp.float8_e4m3fn),
            pltpu.VMEM((MC, N), jnp.bfloat16),
            pltpu.VMEM((N_DEV, 128), jnp.float32),
            pltpu.SemaphoreType.DMA,
            pltpu.SemaphoreType.DMA,
            pltpu.SemaphoreType.DMA((2,)),
            pltpu.SemaphoreType.DMA((2,)),
            pltpu.SemaphoreType.DMA((2,)),
            pltpu.SemaphoreType.DMA((2,)),
            pltpu.SemaphoreType.DMA((N_DEV,)),
            pltpu.SemaphoreType.DMA((N_DEV,)),
            pltpu.SemaphoreType.REGULAR,
            pltpu.SemaphoreType.REGULAR,
        ],
        compiler_params=pltpu.CompilerParams(
            collective_id=0, vmem_limit_bytes=100 * 1024 * 1024
        ),
    )(p)


# device time: 885263 ns/iter; 1.7375x vs baseline; 1.7375x over previous
import jax
import jax.numpy as jnp
from jax import lax
from jax.experimental import pallas as pl
from jax.experimental.pallas import tpu as pltpu

N_DEV = 16
M = 4096
N = 8192
NH = N // 2
MC = M // N_DEV
B_BF16 = 10
MESH = pltpu.DeviceIdType.MESH


class Dir:

    def __init__(self, col0, out_peer, in_peer, wbf, wf32, ag, pc, st,
                 rs_send, rs_recv, ag_send, ag_recv, load_sems, store_sems,
                 rs_cred, ag_cred, recv_chunk, own_row, ag_arrival):
        self.col0 = col0
        self.out_peer = out_peer
        self.in_peer = in_peer
        self.wbf = wbf
        self.wf32 = wf32
        self.ag = ag
        self.pc = pc
        self.st = st
        self.rs_send = rs_send
        self.rs_recv = rs_recv
        self.ag_send = ag_send
        self.ag_recv = ag_recv
        self.load_sems = load_sems
        self.store_sems = store_sems
        self.rs_cred = rs_cred
        self.ag_cred = ag_cred
        self.recv_chunk = recv_chunk
        self.own_row = own_row
        self.ag_arrival = ag_arrival
        self.sends = []
        self.agsends = []
        self.pending_store = [None, None]
        self.acc = None


def kernel(x, w_mat):
    p = jnp.dot(
        x.astype(jnp.bfloat16),
        w_mat.astype(jnp.bfloat16),
        preferred_element_type=jnp.float32,
    )

    def body(p_ref, out_ref, *sc):
        (wa_bf, wa_f32, aga, pca, sta,
         wb_bf, wb_f32, agb, pcb, stb,
         amax_buf,
         a_rs_send, a_rs_recv, a_ag_send, a_ag_recv, a_load, a_store,
         b_rs_send, b_rs_recv, b_ag_send, b_ag_recv, b_load, b_store,
         am_send_sems, am_recv_sems,
         a_rs_cred, a_ag_cred, b_rs_cred, b_ag_cred) = sc

        d = lax.axis_index("i")
        left = (d - 1) % N_DEV
        right = (d + 1) % N_DEV

        barrier = pltpu.get_barrier_semaphore()
        for nbr in (left, right):
            pltpu.semaphore_signal(barrier, 1, device_id=(nbr,), device_id_type=MESH)
        pltpu.semaphore_wait(barrier, 2)

        A = Dir(0, right, left, wa_bf, wa_f32, aga, pca, sta,
                a_rs_send, a_rs_recv, a_ag_send, a_ag_recv, a_load, a_store,
                a_rs_cred, a_ag_cred,
                recv_chunk=lambda h: (d - h - 1) % N_DEV,
                own_row=(d + 1) % N_DEV,
                ag_arrival=lambda g: (d - g) % N_DEV)
        B = Dir(NH, left, right, wb_bf, wb_f32, agb, pcb, stb,
                b_rs_send, b_rs_recv, b_ag_send, b_ag_recv, b_load, b_store,
                b_rs_cred, b_ag_cred,
                recv_chunk=lambda h: (d + h + 1) % N_DEV,
                own_row=(d - 1) % N_DEV,
                ag_arrival=lambda g: (d + g) % N_DEV)
        DIRS = (A, B)

        def load_p(X, chunk_idx, slot):
            cp = pltpu.make_async_copy(
                p_ref.at[pl.ds(chunk_idx * MC, MC), pl.ds(X.col0, NH)],
                X.pc.at[slot],
                X.load_sems.at[slot],
            )
            cp.start()
            return cp

        def fam(X, h):
            return X.wbf if h < B_BF16 else X.wf32

        for X in DIRS:
            load_p(X, d, 0)
        for X in DIRS:
            pltpu.make_async_copy(X.pc.at[0], X.pc.at[0], X.load_sems.at[0]).wait()
            X.wbf[0] = X.pc[0].astype(jnp.bfloat16)
            load_p(X, X.recv_chunk(0), 1)

        for h in range(N_DEV - 1):
            s = h % 2
            r = (h + 1) % 2
            for X in DIRS:
                if h >= 2:
                    pltpu.semaphore_wait(X.rs_cred, 1)
                rd = pltpu.make_async_remote_copy(
                    src_ref=fam(X, h).at[s],
                    dst_ref=fam(X, h).at[r],
                    send_sem=X.rs_send.at[s],
                    recv_sem=X.rs_recv.at[r],
                    device_id=(X.out_peer,),
                    device_id_type=MESH,
                )
                rd.start()
                X.sends.append(rd)
            for X in DIRS:
                if h < N_DEV - 2:
                    load_p(X, X.recv_chunk(h + 1), h % 2)
            for X in DIRS:
                X.sends[h].wait_recv()
                pltpu.make_async_copy(
                    X.pc.at[(h + 1) % 2], X.pc.at[(h + 1) % 2],
                    X.load_sems.at[(h + 1) % 2],
                ).wait()
                recv = fam(X, h)[r]
                if h < B_BF16:
                    recv = recv.astype(jnp.float32)
                acc = recv + X.pc[(h + 1) % 2]
                if h < N_DEV - 2:
                    if (h + 1) < B_BF16:
                        fam(X, h + 1)[r] = acc.astype(jnp.bfloat16)
                    else:
                        fam(X, h + 1)[r] = acc
                    X.sends[h].wait_send()
                else:
                    X.amax = jnp.max(jnp.abs(acc))
                    X.sends[h].wait_send()
                    X.wf32[0] = acc
                if 1 <= h <= 13:
                    pltpu.semaphore_signal(X.rs_cred, 1, device_id=(X.in_peer,),
                                           device_id_type=MESH)

        local_amax = jnp.maximum(A.amax, B.amax)
        amax_buf[pl.ds(d, 1), :] = jnp.full((1, 128), local_amax, jnp.float32)
        am_rdmas = []
        for j in range(1, N_DEV):
            tgt = (d + j) % N_DEV
            rd = pltpu.make_async_remote_copy(
                src_ref=amax_buf.at[pl.ds(d, 1), :],
                dst_ref=amax_buf.at[pl.ds(d, 1), :],
                send_sem=am_send_sems.at[j],
                recv_sem=am_recv_sems.at[j],
                device_id=(tgt,),
                device_id_type=MESH,
            )
            rd.start()
            am_rdmas.append(rd)
        for j in range(1, N_DEV):
            src_dev = (d - j) % N_DEV
            pltpu.make_async_remote_copy(
                src_ref=amax_buf.at[pl.ds(src_dev, 1), :],
                dst_ref=amax_buf.at[pl.ds(src_dev, 1), :],
                send_sem=am_send_sems.at[j],
                recv_sem=am_recv_sems.at[j],
                device_id=(src_dev,),
                device_id_type=MESH,
            ).wait_recv()
        for rd in am_rdmas:
            rd.wait_send()
        amax = jnp.max(amax_buf[:, :])
        scale = amax / 448.0
        inv_scale = 448.0 / amax

        for X in DIRS:
            X.ag[0] = (X.wf32[0][:, :] * inv_scale).astype(jnp.float8_e4m3fn)
            X.st[0] = (X.ag[0].astype(jnp.float32) * scale).astype(jnp.bfloat16)
            cp = pltpu.make_async_copy(
                X.st.at[0],
                out_ref.at[pl.ds(X.own_row * MC, MC), pl.ds(X.col0, NH)],
                X.store_sems.at[0],
            )
            cp.start()
            X.pending_store[0] = cp

        for g in range(N_DEV - 1):
            s = g % 2
            r = (g + 1) % 2
            for X in DIRS:
                if g >= 2:
                    pltpu.semaphore_wait(X.ag_cred, 1)
                rd = pltpu.make_async_remote_copy(
                    src_ref=X.ag.at[s],
                    dst_ref=X.ag.at[r],
                    send_sem=X.ag_send.at[s],
                    recv_sem=X.ag_recv.at[r],
                    device_id=(X.out_peer,),
                    device_id_type=MESH,
                )
                rd.start()
                X.agsends.append(rd)
            for X in DIRS:
                X.agsends[g].wait_recv()
                st_slot = (g + 1) % 2
                if X.pending_store[st_slot] is not None:
                    X.pending_store[st_slot].wait()
                X.st[st_slot] = (
                    X.ag[r].astype(jnp.float32) * scale
                ).astype(jnp.bfloat16)
                cp = pltpu.make_async_copy(
                    X.st.at[st_slot],
                    out_ref.at[pl.ds(X.ag_arrival(g) * MC, MC), pl.ds(X.col0, NH)],
                    X.store_sems.at[st_slot],
                )
                cp.start()
                X.pending_store[st_slot] = cp
                X.agsends[g].wait_send()
                if 1 <= g <= 13:
                    pltpu.semaphore_signal(X.ag_cred, 1, device_id=(X.in_peer,),
                                           device_id_type=MESH)

        for X in DIRS:
            for cp in X.pending_store:
                if cp is not None:
                    cp.wait()

    out_shape = jax.ShapeDtypeStruct((M, N), jnp.bfloat16)
    return pl.pallas_call(
        body,
        out_shape=out_shape,
        in_specs=[pl.BlockSpec(memory_space=pl.ANY)],
        out_specs=pl.BlockSpec(memory_space=pl.ANY),
        scratch_shapes=[
            pltpu.VMEM((2, MC, NH), jnp.bfloat16),
            pltpu.VMEM((2, MC, NH), jnp.float32),
            pltpu.VMEM((2, MC, NH), jnp.float8_e4m3fn),
            pltpu.VMEM((2, MC, NH), jnp.float32),
            pltpu.VMEM((2, MC, NH), jnp.bfloat16),
            pltpu.VMEM((2, MC, NH), jnp.bfloat16),
            pltpu.VMEM((2, MC, NH), jnp.float32),
            pltpu.VMEM((2, MC, NH), jnp.float8_e4m3fn),
            pltpu.VMEM((2, MC, NH), jnp.float32),
            pltpu.VMEM((2, MC, NH), jnp.bfloat16),
            pltpu.VMEM((N_DEV, 128), jnp.float32),
            pltpu.SemaphoreType.DMA((2,)),
            pltpu.SemaphoreType.DMA((2,)),
            pltpu.SemaphoreType.DMA((2,)),
            pltpu.SemaphoreType.DMA((2,)),
            pltpu.SemaphoreType.DMA((2,)),
            pltpu.SemaphoreType.DMA((2,)),
            pltpu.SemaphoreType.DMA((2,)),
            pltpu.SemaphoreType.DMA((2,)),
            pltpu.SemaphoreType.DMA((2,)),
            pltpu.SemaphoreType.DMA((2,)),
            pltpu.SemaphoreType.DMA((2,)),
            pltpu.SemaphoreType.DMA((2,)),
            pltpu.SemaphoreType.DMA((N_DEV,)),
            pltpu.SemaphoreType.DMA((N_DEV,)),
            pltpu.SemaphoreType.REGULAR,
            pltpu.SemaphoreType.REGULAR,
            pltpu.SemaphoreType.REGULAR,
            pltpu.SemaphoreType.REGULAR,
        ],
        compiler_params=pltpu.CompilerParams(
            collective_id=0, vmem_limit_bytes=100 * 1024 * 1024
        ),
    )(p)


# device time: 827414 ns/iter; 1.8590x vs baseline; 1.0699x over previous
import jax
import jax.numpy as jnp
from jax import lax
from jax.experimental import pallas as pl
from jax.experimental.pallas import tpu as pltpu

N_DEV = 16
M = 4096
N = 8192
N_LANE = 4
NL = N // N_LANE
MC = M // N_DEV
B_BF16 = 10
MESH = pltpu.DeviceIdType.MESH

_LANE_SCRATCH = [
    pltpu.VMEM((2, MC, NL), jnp.bfloat16),
    pltpu.VMEM((2, MC, NL), jnp.float32),
    pltpu.VMEM((2, MC, NL), jnp.float8_e4m3fn),
    pltpu.VMEM((2, MC, NL), jnp.float32),
    pltpu.VMEM((2, MC, NL), jnp.bfloat16),
    pltpu.SemaphoreType.DMA((2,)),
    pltpu.SemaphoreType.DMA((2,)),
    pltpu.SemaphoreType.DMA((2,)),
    pltpu.SemaphoreType.DMA((2,)),
    pltpu.SemaphoreType.DMA((2,)),
    pltpu.SemaphoreType.DMA((2,)),
    pltpu.SemaphoreType.REGULAR,
    pltpu.SemaphoreType.REGULAR,
]
_PER_LANE = len(_LANE_SCRATCH)


class Lane:
    def __init__(self, refs, col0, out_peer, in_peer,
                 recv_chunk, own_row, ag_arrival):
        (self.wbf, self.wf32, self.ag, self.pc, self.st,
         self.rs_send, self.rs_recv, self.ag_send, self.ag_recv,
         self.load_sems, self.store_sems, self.rs_cred, self.ag_cred) = refs
        self.col0 = col0
        self.out_peer = out_peer
        self.in_peer = in_peer
        self.recv_chunk = recv_chunk
        self.own_row = own_row
        self.ag_arrival = ag_arrival
        self.sends = []
        self.agsends = []
        self.pending_store = [None, None]
        self.amax = None


def kernel(x, w_mat):
    p = jnp.dot(
        x.astype(jnp.bfloat16),
        w_mat.astype(jnp.bfloat16),
        preferred_element_type=jnp.float32,
    )

    def body(p_ref, out_ref, *sc):
        amax_buf, am_send_sems, am_recv_sems = sc[N_LANE * _PER_LANE:]

        d = lax.axis_index("i")
        left = (d - 1) % N_DEV
        right = (d + 1) % N_DEV

        barrier = pltpu.get_barrier_semaphore()
        for nbr in (left, right):
            pltpu.semaphore_signal(barrier, 1, device_id=(nbr,), device_id_type=MESH)
        pltpu.semaphore_wait(barrier, 2)

        def mk_lane(i):
            refs = sc[i * _PER_LANE:(i + 1) * _PER_LANE]
            rightward = i < 2
            if rightward:
                return Lane(refs, i * NL, right, left,
                            recv_chunk=lambda h: (d - h - 1) % N_DEV,
                            own_row=(d + 1) % N_DEV,
                            ag_arrival=lambda g: (d - g) % N_DEV)
            return Lane(refs, i * NL, left, right,
                        recv_chunk=lambda h: (d + h + 1) % N_DEV,
                        own_row=(d - 1) % N_DEV,
                        ag_arrival=lambda g: (d + g) % N_DEV)

        LANES = [mk_lane(0), mk_lane(2), mk_lane(1), mk_lane(3)]

        def load_p(X, chunk_idx, slot):
            cp = pltpu.make_async_copy(
                p_ref.at[pl.ds(chunk_idx * MC, MC), pl.ds(X.col0, NL)],
                X.pc.at[slot],
                X.load_sems.at[slot],
            )
            cp.start()
            return cp

        def wait_load(X, slot):
            pltpu.make_async_copy(
                X.pc.at[slot], X.pc.at[slot], X.load_sems.at[slot]
            ).wait()

        def fam(X, h):
            return X.wbf if h < B_BF16 else X.wf32

        for X in LANES:
            load_p(X, d, 0)
        for X in LANES:
            wait_load(X, 0)
            X.wbf[0] = X.pc[0].astype(jnp.bfloat16)
            load_p(X, X.recv_chunk(0), 1)

        for h in range(N_DEV - 1):
            s = h % 2
            r = (h + 1) % 2
            for X in LANES:
                if h >= 2:
                    pltpu.semaphore_wait(X.rs_cred, 1)
                rd = pltpu.make_async_remote_copy(
                    src_ref=fam(X, h).at[s],
                    dst_ref=fam(X, h).at[r],
                    send_sem=X.rs_send.at[s],
                    recv_sem=X.rs_recv.at[r],
                    device_id=(X.out_peer,),
                    device_id_type=MESH,
                )
                rd.start()
                X.sends.append(rd)
            for X in LANES:
                if h < N_DEV - 2:
                    load_p(X, X.recv_chunk(h + 1), h % 2)
            for X in LANES:
                X.sends[h].wait_recv()
                wait_load(X, (h + 1) % 2)
                recv = fam(X, h)[r].astype(jnp.float32)
                acc = recv + X.pc[(h + 1) % 2]
                if h < N_DEV - 2:
                    if (h + 1) < B_BF16:
                        fam(X, h + 1)[r] = acc.astype(jnp.bfloat16)
                    else:
                        fam(X, h + 1)[r] = acc
                    X.sends[h].wait_send()
                else:
                    X.amax = jnp.max(jnp.abs(acc))
                    X.sends[h].wait_send()
                    X.wf32[0] = acc
                if 1 <= h <= 13:
                    pltpu.semaphore_signal(X.rs_cred, 1, device_id=(X.in_peer,),
                                           device_id_type=MESH)

        local_amax = jnp.max(jnp.stack([X.amax for X in LANES]))
        amax_buf[pl.ds(d, 1), :] = jnp.full((1, 128), local_amax, jnp.float32)
        am_rdmas = []
        for j in range(1, N_DEV):
            tgt = (d + j) % N_DEV
            rd = pltpu.make_async_remote_copy(
                src_ref=amax_buf.at[pl.ds(d, 1), :],
                dst_ref=amax_buf.at[pl.ds(d, 1), :],
                send_sem=am_send_sems.at[j],
                recv_sem=am_recv_sems.at[j],
                device_id=(tgt,),
                device_id_type=MESH,
            )
            rd.start()
            am_rdmas.append(rd)
        for j in range(1, N_DEV):
            src_dev = (d - j) % N_DEV
            pltpu.make_async_remote_copy(
                src_ref=amax_buf.at[pl.ds(src_dev, 1), :],
                dst_ref=amax_buf.at[pl.ds(src_dev, 1), :],
                send_sem=am_send_sems.at[j],
                recv_sem=am_recv_sems.at[j],
                device_id=(src_dev,),
                device_id_type=MESH,
            ).wait_recv()
        for rd in am_rdmas:
            rd.wait_send()
        amax = jnp.max(amax_buf[:, :])
        scale = amax / 448.0
        inv_scale = 448.0 / amax

        def stage_out(X, st_slot, src_slot, chunk_idx):
            if X.pending_store[st_slot] is not None:
                X.pending_store[st_slot].wait()
            X.st[st_slot] = (
                X.ag[src_slot].astype(jnp.float32) * scale
            ).astype(jnp.bfloat16)
            cp = pltpu.make_async_copy(
                X.st.at[st_slot],
                out_ref.at[pl.ds(chunk_idx * MC, MC), pl.ds(X.col0, NL)],
                X.store_sems.at[st_slot],
            )
            cp.start()
            X.pending_store[st_slot] = cp

        for X in LANES:
            X.ag[0] = (X.wf32[0][:, :] * inv_scale).astype(jnp.float8_e4m3fn)
            stage_out(X, 0, 0, X.own_row)

        for g in range(N_DEV - 1):
            s = g % 2
            r = (g + 1) % 2
            for X in LANES:
                if g >= 2:
                    pltpu.semaphore_wait(X.ag_cred, 1)
                rd = pltpu.make_async_remote_copy(
                    src_ref=X.ag.at[s],
                    dst_ref=X.ag.at[r],
                    send_sem=X.ag_send.at[s],
                    recv_sem=X.ag_recv.at[r],
                    device_id=(X.out_peer,),
                    device_id_type=MESH,
                )
                rd.start()
                X.agsends.append(rd)
            for X in LANES:
                X.agsends[g].wait_recv()
                stage_out(X, (g + 1) % 2, r, X.ag_arrival(g))
                X.agsends[g].wait_send()
                if 1 <= g <= 13:
                    pltpu.semaphore_signal(X.ag_cred, 1, device_id=(X.in_peer,),
                                           device_id_type=MESH)

        for X in LANES:
            for cp in X.pending_store:
                if cp is not None:
                    cp.wait()

    out_shape = jax.ShapeDtypeStruct((M, N), jnp.bfloat16)
    return pl.pallas_call(
        body,
        out_shape=out_shape,
        in_specs=[pl.BlockSpec(memory_space=pl.ANY)],
        out_specs=pl.BlockSpec(memory_space=pl.ANY),
        scratch_shapes=(
            _LANE_SCRATCH * N_LANE
            + [
                pltpu.VMEM((N_DEV, 128), jnp.float32),
                pltpu.SemaphoreType.DMA((N_DEV,)),
                pltpu.SemaphoreType.DMA((N_DEV,)),
            ]
        ),
        compiler_params=pltpu.CompilerParams(
            collective_id=0, vmem_limit_bytes=100 * 1024 * 1024
        ),
    )(p)


# device time: 791303 ns/iter; 1.9438x vs baseline; 1.0456x over previous
import jax
import jax.numpy as jnp
from jax import lax
from jax.experimental import pallas as pl
from jax.experimental.pallas import tpu as pltpu

N_DEV = 16
M = 4096
K = 256
N = 8192
N_LANE = 4
NL = N // N_LANE
MC = M // N_DEV
B_BF16 = 10
MESH = pltpu.DeviceIdType.MESH

_LANE_SCRATCH = [
    pltpu.VMEM((2, MC, NL), jnp.bfloat16),
    pltpu.VMEM((2, MC, NL), jnp.float32),
    pltpu.VMEM((2, MC, NL), jnp.float8_e4m3fn),
    pltpu.VMEM((2, MC, NL), jnp.bfloat16),
    pltpu.SemaphoreType.DMA((2,)),
    pltpu.SemaphoreType.DMA((2,)),
    pltpu.SemaphoreType.DMA((2,)),
    pltpu.SemaphoreType.DMA((2,)),
    pltpu.SemaphoreType.DMA((2,)),
    pltpu.SemaphoreType.REGULAR,
    pltpu.SemaphoreType.REGULAR,
]
_PER_LANE = len(_LANE_SCRATCH)


class Lane:
    def __init__(self, refs, col0, out_peer, in_peer,
                 recv_chunk, own_row, ag_arrival):
        (self.wbf, self.wf32, self.ag, self.st,
         self.rs_send, self.rs_recv, self.ag_send, self.ag_recv,
         self.store_sems, self.rs_cred, self.ag_cred) = refs
        self.col0 = col0
        self.out_peer = out_peer
        self.in_peer = in_peer
        self.recv_chunk = recv_chunk
        self.own_row = own_row
        self.ag_arrival = ag_arrival
        self.sends = []
        self.agsends = []
        self.pending_store = [None, None]
        self.amax = None


def kernel(x, w_mat):
    x16 = x.astype(jnp.bfloat16)
    w16 = w_mat.astype(jnp.bfloat16)

    def body(x_ref, w_ref, out_ref, *sc):
        amax_buf, am_send_sems, am_recv_sems = sc[N_LANE * _PER_LANE:]

        d = lax.axis_index("i")
        left = (d - 1) % N_DEV
        right = (d + 1) % N_DEV

        barrier = pltpu.get_barrier_semaphore()
        for nbr in (left, right):
            pltpu.semaphore_signal(barrier, 1, device_id=(nbr,), device_id_type=MESH)
        pltpu.semaphore_wait(barrier, 2)

        def mk_lane(i):
            refs = sc[i * _PER_LANE:(i + 1) * _PER_LANE]
            rightward = i < 2
            if rightward:
                return Lane(refs, i * NL, right, left,
                            recv_chunk=lambda h: (d - h - 1) % N_DEV,
                            own_row=(d + 1) % N_DEV,
                            ag_arrival=lambda g: (d - g) % N_DEV)
            return Lane(refs, i * NL, left, right,
                        recv_chunk=lambda h: (d + h + 1) % N_DEV,
                        own_row=(d - 1) % N_DEV,
                        ag_arrival=lambda g: (d + g) % N_DEV)

        LANES = [mk_lane(0), mk_lane(2), mk_lane(1), mk_lane(3)]

        def partial(X, chunk_idx):
            return jnp.dot(
                x_ref[pl.ds(chunk_idx * MC, MC), :],
                w_ref[:, pl.ds(X.col0, NL)],
                preferred_element_type=jnp.float32,
            )

        def fam(X, h):
            return X.wbf if h < B_BF16 else X.wf32

        for X in LANES:
            X.wbf[0] = partial(X, d).astype(jnp.bfloat16)

        for h in range(N_DEV - 1):
            s = h % 2
            r = (h + 1) % 2
            for X in LANES:
                if h >= 1:
                    pltpu.semaphore_wait(X.rs_cred, 1)
                rd = pltpu.make_async_remote_copy(
                    src_ref=fam(X, h).at[s],
                    dst_ref=fam(X, h).at[r],
                    send_sem=X.rs_send.at[s],
                    recv_sem=X.rs_recv.at[r],
                    device_id=(X.out_peer,),
                    device_id_type=MESH,
                )
                rd.start()
                X.sends.append(rd)
            for X in LANES:
                X.sends[h].wait_recv()
                recv = fam(X, h)[r].astype(jnp.float32)
                acc = recv + partial(X, X.recv_chunk(h))
                if h < N_DEV - 2:
                    if (h + 1) < B_BF16:
                        fam(X, h + 1)[r] = acc.astype(jnp.bfloat16)
                    else:
                        fam(X, h + 1)[r] = acc
                    X.sends[h].wait_send()
                else:
                    X.amax = jnp.max(jnp.abs(acc))
                    X.sends[h].wait_send()
                    X.wf32[0] = acc
                if h <= 13:
                    pltpu.semaphore_signal(X.rs_cred, 1, device_id=(X.in_peer,),
                                           device_id_type=MESH)

        local_amax = jnp.max(jnp.stack([X.amax for X in LANES]))
        amax_buf[pl.ds(d, 1), :] = jnp.full((1, 128), local_amax, jnp.float32)
        am_rdmas = []
        for j in range(1, N_DEV):
            tgt = (d + j) % N_DEV
            rd = pltpu.make_async_remote_copy(
                src_ref=amax_buf.at[pl.ds(d, 1), :],
                dst_ref=amax_buf.at[pl.ds(d, 1), :],
                send_sem=am_send_sems.at[j],
                recv_sem=am_recv_sems.at[j],
                device_id=(tgt,),
                device_id_type=MESH,
            )
            rd.start()
            am_rdmas.append(rd)
        for j in range(1, N_DEV):
            src_dev = (d - j) % N_DEV
            pltpu.make_async_remote_copy(
                src_ref=amax_buf.at[pl.ds(src_dev, 1), :],
                dst_ref=amax_buf.at[pl.ds(src_dev, 1), :],
                send_sem=am_send_sems.at[j],
                recv_sem=am_recv_sems.at[j],
                device_id=(src_dev,),
                device_id_type=MESH,
            ).wait_recv()
        for rd in am_rdmas:
            rd.wait_send()
        amax = jnp.max(amax_buf[:, :])
        scale = amax / 448.0
        inv_scale = 448.0 / amax

        def stage_out(X, st_slot, src_slot, chunk_idx):
            if X.pending_store[st_slot] is not None:
                X.pending_store[st_slot].wait()
            X.st[st_slot] = (
                X.ag[src_slot].astype(jnp.float32) * scale
            ).astype(jnp.bfloat16)
            cp = pltpu.make_async_copy(
                X.st.at[st_slot],
                out_ref.at[pl.ds(chunk_idx * MC, MC), pl.ds(X.col0, NL)],
                X.store_sems.at[st_slot],
            )
            cp.start()
            X.pending_store[st_slot] = cp

        for X in LANES:
            X.ag[0] = (X.wf32[0][:, :] * inv_scale).astype(jnp.float8_e4m3fn)

        for g in range(N_DEV - 1):
            s = g % 2
            r = (g + 1) % 2
            for X in LANES:
                if g >= 1:
                    pltpu.semaphore_wait(X.ag_cred, 1)
                rd = pltpu.make_async_remote_copy(
                    src_ref=X.ag.at[s],
                    dst_ref=X.ag.at[r],
                    send_sem=X.ag_send.at[s],
                    recv_sem=X.ag_recv.at[r],
                    device_id=(X.out_peer,),
                    device_id_type=MESH,
                )
                rd.start()
                X.agsends.append(rd)
            for X in LANES:
                if g == 0:
                    stage_out(X, 0, 0, X.own_row)
                else:
                    stage_out(X, g % 2, s, X.ag_arrival(g - 1))
            for X in LANES:
                X.agsends[g].wait_recv()
                X.agsends[g].wait_send()
                if g <= 13:
                    pltpu.semaphore_signal(X.ag_cred, 1, device_id=(X.in_peer,),
                                           device_id_type=MESH)

        for X in LANES:
            stage_out(X, 1, 1, X.ag_arrival(N_DEV - 2))
        for X in LANES:
            for cp in X.pending_store:
                if cp is not None:
                    cp.wait()

    out_shape = jax.ShapeDtypeStruct((M, N), jnp.bfloat16)
    return pl.pallas_call(
        body,
        out_shape=out_shape,
        in_specs=[
            pl.BlockSpec(memory_space=pltpu.VMEM),
            pl.BlockSpec(memory_space=pltpu.VMEM),
        ],
        out_specs=pl.BlockSpec(memory_space=pl.ANY),
        scratch_shapes=(
            _LANE_SCRATCH * N_LANE
            + [
                pltpu.VMEM((N_DEV, 128), jnp.float32),
                pltpu.SemaphoreType.DMA((N_DEV,)),
                pltpu.SemaphoreType.DMA((N_DEV,)),
            ]
        ),
        compiler_params=pltpu.CompilerParams(
            collective_id=0, vmem_limit_bytes=100 * 1024 * 1024
        ),
    )(x16, w16)


# device time: 689857 ns/iter; 2.2297x vs baseline; 1.1471x over previous
import jax
import jax.numpy as jnp
from jax import lax
from jax.experimental import pallas as pl
from jax.experimental.pallas import tpu as pltpu

N_DEV = 16
M = 4096
K = 256
N = 8192
N_LANE = 4
NL = N // N_LANE
MC = M // N_DEV
B_BF16 = 10
MESH = pltpu.DeviceIdType.MESH

_LANE_SCRATCH = [
    pltpu.VMEM((2, MC, NL), jnp.bfloat16),
    pltpu.VMEM((2, MC, NL), jnp.float32),
    pltpu.VMEM((3, MC, NL), jnp.float8_e4m3fn),
    pltpu.VMEM((2, MC, NL), jnp.bfloat16),
    pltpu.SemaphoreType.DMA((2,)),
    pltpu.SemaphoreType.DMA((2,)),
    pltpu.SemaphoreType.DMA((3,)),
    pltpu.SemaphoreType.DMA((3,)),
    pltpu.SemaphoreType.DMA((2,)),
    pltpu.SemaphoreType.REGULAR,
    pltpu.SemaphoreType.REGULAR,
]
_PER_LANE = len(_LANE_SCRATCH)


class Lane:
    def __init__(self, refs, col0, out_peer, in_peer,
                 recv_chunk, own_row, ag_arrival):
        (self.wbf, self.wf32, self.ag, self.st,
         self.rs_send, self.rs_recv, self.ag_send, self.ag_recv,
         self.store_sems, self.rs_cred, self.ag_cred) = refs
        self.col0 = col0
        self.out_peer = out_peer
        self.in_peer = in_peer
        self.recv_chunk = recv_chunk
        self.own_row = own_row
        self.ag_arrival = ag_arrival
        self.sends = []
        self.agsends = []
        self.pending_store = [None, None]
        self.amax = None


def kernel(x, w_mat):
    x16 = x.astype(jnp.bfloat16)
    w16 = w_mat.astype(jnp.bfloat16)

    def body(x_ref, w_ref, out_ref, *sc):
        amax_buf, am_send_sems, am_recv_sems = sc[N_LANE * _PER_LANE:]

        d = lax.axis_index("i")
        left = (d - 1) % N_DEV
        right = (d + 1) % N_DEV

        barrier = pltpu.get_barrier_semaphore()
        for nbr in (left, right):
            pltpu.semaphore_signal(barrier, 1, device_id=(nbr,), device_id_type=MESH)
        pltpu.semaphore_wait(barrier, 2)

        def mk_lane(i):
            refs = sc[i * _PER_LANE:(i + 1) * _PER_LANE]
            rightward = i < 2
            if rightward:
                return Lane(refs, i * NL, right, left,
                            recv_chunk=lambda h: (d - h - 1) % N_DEV,
                            own_row=(d + 1) % N_DEV,
                            ag_arrival=lambda g: (d - g) % N_DEV)
            return Lane(refs, i * NL, left, right,
                        recv_chunk=lambda h: (d + h + 1) % N_DEV,
                        own_row=(d - 1) % N_DEV,
                        ag_arrival=lambda g: (d + g) % N_DEV)

        LANES = [mk_lane(0), mk_lane(2), mk_lane(1), mk_lane(3)]

        def partial(X, chunk_idx):
            return jnp.dot(
                x_ref[pl.ds(chunk_idx * MC, MC), :],
                w_ref[:, pl.ds(X.col0, NL)],
                preferred_element_type=jnp.float32,
            )

        def fam(X, h):
            return X.wbf if h < B_BF16 else X.wf32

        def rs_send(X, h):
            s = h % 2
            r = (h + 1) % 2
            rd = pltpu.make_async_remote_copy(
                src_ref=fam(X, h).at[s],
                dst_ref=fam(X, h).at[r],
                send_sem=X.rs_send.at[s],
                recv_sem=X.rs_recv.at[r],
                device_id=(X.out_peer,),
                device_id_type=MESH,
            )
            rd.start()
            X.sends.append(rd)

        for X in LANES:
            X.wbf[0] = partial(X, d).astype(jnp.bfloat16)
            rs_send(X, 0)

        for h in range(N_DEV - 1):
            r = (h + 1) % 2
            for X in LANES:
                X.sends[h].wait_recv()
                recv = fam(X, h)[r].astype(jnp.float32)
                acc = recv + partial(X, X.recv_chunk(h))
                if h < N_DEV - 2:
                    if (h + 1) < B_BF16:
                        fam(X, h + 1)[r] = acc.astype(jnp.bfloat16)
                    else:
                        fam(X, h + 1)[r] = acc
                else:
                    X.amax = jnp.max(jnp.abs(acc))
                X.sends[h].wait_send()
                if h == N_DEV - 2:
                    X.wf32[0] = acc
                if h <= 13:
                    pltpu.semaphore_signal(X.rs_cred, 1, device_id=(X.in_peer,),
                                           device_id_type=MESH)
                if h < N_DEV - 2:
                    pltpu.semaphore_wait(X.rs_cred, 1)
                    rs_send(X, h + 1)

        local_amax = jnp.max(jnp.stack([X.amax for X in LANES]))
        amax_buf[pl.ds(d, 1), :] = jnp.full((1, 128), local_amax, jnp.float32)
        am_rdmas = []
        for j in range(1, N_DEV):
            tgt = (d + j) % N_DEV
            rd = pltpu.make_async_remote_copy(
                src_ref=amax_buf.at[pl.ds(d, 1), :],
                dst_ref=amax_buf.at[pl.ds(d, 1), :],
                send_sem=am_send_sems.at[j],
                recv_sem=am_recv_sems.at[j],
                device_id=(tgt,),
                device_id_type=MESH,
            )
            rd.start()
            am_rdmas.append(rd)
        for j in range(1, N_DEV):
            src_dev = (d - j) % N_DEV
            pltpu.make_async_remote_copy(
                src_ref=amax_buf.at[pl.ds(src_dev, 1), :],
                dst_ref=amax_buf.at[pl.ds(src_dev, 1), :],
                send_sem=am_send_sems.at[j],
                recv_sem=am_recv_sems.at[j],
                device_id=(src_dev,),
                device_id_type=MESH,
            ).wait_recv()
        for rd in am_rdmas:
            rd.wait_send()
        amax = jnp.max(amax_buf[:, :])
        scale = amax / 448.0
        inv_scale = 448.0 / amax

        def stage_out(X, st_slot, src_slot, chunk_idx):
            if X.pending_store[st_slot] is not None:
                X.pending_store[st_slot].wait()
            X.st[st_slot] = (
                X.ag[src_slot].astype(jnp.float32) * scale
            ).astype(jnp.bfloat16)
            cp = pltpu.make_async_copy(
                X.st.at[st_slot],
                out_ref.at[pl.ds(chunk_idx * MC, MC), pl.ds(X.col0, NL)],
                X.store_sems.at[st_slot],
            )
            cp.start()
            X.pending_store[st_slot] = cp

        def ag_send(X, g):
            rd = pltpu.make_async_remote_copy(
                src_ref=X.ag.at[g % 3],
                dst_ref=X.ag.at[(g + 1) % 3],
                send_sem=X.ag_send.at[g % 3],
                recv_sem=X.ag_recv.at[(g + 1) % 3],
                device_id=(X.out_peer,),
                device_id_type=MESH,
            )
            rd.start()
            X.agsends.append(rd)

        for X in LANES:
            X.ag[0] = (X.wf32[0][:, :] * inv_scale).astype(jnp.float8_e4m3fn)
            ag_send(X, 0)
        for X in LANES:
            stage_out(X, 0, 0, X.own_row)

        for g in range(N_DEV - 1):
            for X in LANES:
                X.agsends[g].wait_recv()
                if g <= 13:
                    pltpu.semaphore_signal(X.ag_cred, 1, device_id=(X.in_peer,),
                                           device_id_type=MESH)
                if g < N_DEV - 2:
                    pltpu.semaphore_wait(X.ag_cred, 1)
                    ag_send(X, g + 1)
                X.agsends[g].wait_send()
                stage_out(X, (g + 1) % 2, (g + 1) % 3, X.ag_arrival(g))

        for X in LANES:
            for cp in X.pending_store:
                if cp is not None:
                    cp.wait()

    out_shape = jax.ShapeDtypeStruct((M, N), jnp.bfloat16)
    return pl.pallas_call(
        body,
        out_shape=out_shape,
        in_specs=[
            pl.BlockSpec(memory_space=pltpu.VMEM),
            pl.BlockSpec(memory_space=pltpu.VMEM),
        ],
        out_specs=pl.BlockSpec(memory_space=pl.ANY),
        scratch_shapes=(
            _LANE_SCRATCH * N_LANE
            + [
                pltpu.VMEM((N_DEV, 128), jnp.float32),
                pltpu.SemaphoreType.DMA((N_DEV,)),
                pltpu.SemaphoreType.DMA((N_DEV,)),
            ]
        ),
        compiler_params=pltpu.CompilerParams(
            collective_id=0, vmem_limit_bytes=100 * 1024 * 1024
        ),
    )(x16, w16)


# device time: 669292 ns/iter; 2.2982x vs baseline; 1.0307x over previous
import jax
import jax.numpy as jnp
from jax import lax
from jax.experimental import pallas as pl
from jax.experimental.pallas import tpu as pltpu

N_DEV = 16
M = 4096
K = 256
N = 8192
N_LANE = 4
NL = N // N_LANE
MC = M // N_DEV
B_BF16 = 11
MESH = pltpu.DeviceIdType.MESH

_LANE_SCRATCH = [
    pltpu.VMEM((2, MC, NL), jnp.bfloat16),
    pltpu.VMEM((2, MC, NL), jnp.float32),
    pltpu.VMEM((3, MC, NL), jnp.float8_e4m3fn),
    pltpu.VMEM((2, MC, NL), jnp.bfloat16),
    pltpu.SemaphoreType.DMA((2,)),
    pltpu.SemaphoreType.DMA((2,)),
    pltpu.SemaphoreType.DMA((3,)),
    pltpu.SemaphoreType.DMA((3,)),
    pltpu.SemaphoreType.DMA((2,)),
    pltpu.SemaphoreType.REGULAR,
    pltpu.SemaphoreType.REGULAR,
]
_PER_LANE = len(_LANE_SCRATCH)


class Lane:
    def __init__(self, refs, col0, out_peer, in_peer,
                 recv_chunk, own_row, ag_arrival):
        (self.wbf, self.wf32, self.ag, self.st,
         self.rs_send, self.rs_recv, self.ag_send, self.ag_recv,
         self.store_sems, self.rs_cred, self.ag_cred) = refs
        self.col0 = col0
        self.out_peer = out_peer
        self.in_peer = in_peer
        self.recv_chunk = recv_chunk
        self.own_row = own_row
        self.ag_arrival = ag_arrival
        self.sends = []
        self.agsends = []
        self.pending_store = [None, None]
        self.amax = None


def kernel(x, w_mat):
    x16 = x.astype(jnp.bfloat16)
    w16 = w_mat.astype(jnp.bfloat16)

    def body(x_ref, w_ref, out_ref, *sc):
        amax_buf, am_send_sems, am_recv_sems = sc[N_LANE * _PER_LANE:]

        d = lax.axis_index("i")
        left = (d - 1) % N_DEV
        right = (d + 1) % N_DEV

        barrier = pltpu.get_barrier_semaphore()
        for nbr in (left, right):
            pltpu.semaphore_signal(barrier, 1, device_id=(nbr,), device_id_type=MESH)
        pltpu.semaphore_wait(barrier, 2)

        def mk_lane(i):
            refs = sc[i * _PER_LANE:(i + 1) * _PER_LANE]
            rightward = i < 2
            if rightward:
                return Lane(refs, i * NL, right, left,
                            recv_chunk=lambda h: (d - h - 1) % N_DEV,
                            own_row=(d + 1) % N_DEV,
                            ag_arrival=lambda g: (d - g) % N_DEV)
            return Lane(refs, i * NL, left, right,
                        recv_chunk=lambda h: (d + h + 1) % N_DEV,
                        own_row=(d - 1) % N_DEV,
                        ag_arrival=lambda g: (d + g) % N_DEV)

        LANES = [mk_lane(0), mk_lane(2), mk_lane(1), mk_lane(3)]

        def partial(X, chunk_idx):
            return jnp.dot(
                x_ref[pl.ds(chunk_idx * MC, MC), :],
                w_ref[:, pl.ds(X.col0, NL)],
                preferred_element_type=jnp.float32,
            )

        def fam(X, h):
            return X.wbf if h < B_BF16 else X.wf32

        def rs_send(X, h):
            s = h % 2
            r = (h + 1) % 2
            rd = pltpu.make_async_remote_copy(
                src_ref=fam(X, h).at[s],
                dst_ref=fam(X, h).at[r],
                send_sem=X.rs_send.at[s],
                recv_sem=X.rs_recv.at[r],
                device_id=(X.out_peer,),
                device_id_type=MESH,
            )
            rd.start()
            X.sends.append(rd)

        for X in LANES:
            X.wbf[0] = partial(X, d).astype(jnp.bfloat16)
            rs_send(X, 0)

        for h in range(N_DEV - 1):
            r = (h + 1) % 2
            for X in LANES:
                X.sends[h].wait_recv()
                recv = fam(X, h)[r].astype(jnp.float32)
                acc = recv + partial(X, X.recv_chunk(h))
                if h < N_DEV - 2:
                    if (h + 1) < B_BF16:
                        fam(X, h + 1)[r] = acc.astype(jnp.bfloat16)
                    else:
                        fam(X, h + 1)[r] = acc
                else:
                    X.amax = jnp.max(jnp.abs(acc))
                X.sends[h].wait_send()
                if h == N_DEV - 2:
                    X.wf32[0] = acc
                if h <= 13:
                    pltpu.semaphore_signal(X.rs_cred, 1, device_id=(X.in_peer,),
                                           device_id_type=MESH)
                if h < N_DEV - 2:
                    pltpu.semaphore_wait(X.rs_cred, 1)
                    rs_send(X, h + 1)

        local_amax = jnp.max(jnp.stack([X.amax for X in LANES]))
        amax_buf[pl.ds(d, 1), :] = jnp.full((1, 128), local_amax, jnp.float32)
        am_rdmas = []
        for j in range(1, N_DEV):
            tgt = (d + j) % N_DEV
            rd = pltpu.make_async_remote_copy(
                src_ref=amax_buf.at[pl.ds(d, 1), :],
                dst_ref=amax_buf.at[pl.ds(d, 1), :],
                send_sem=am_send_sems.at[j],
                recv_sem=am_recv_sems.at[j],
                device_id=(tgt,),
                device_id_type=MESH,
            )
            rd.start()
            am_rdmas.append(rd)
        for j in range(1, N_DEV):
            src_dev = (d - j) % N_DEV
            pltpu.make_async_remote_copy(
                src_ref=amax_buf.at[pl.ds(src_dev, 1), :],
                dst_ref=amax_buf.at[pl.ds(src_dev, 1), :],
                send_sem=am_send_sems.at[j],
                recv_sem=am_recv_sems.at[j],
                device_id=(src_dev,),
                device_id_type=MESH,
            ).wait_recv()
        for rd in am_rdmas:
            rd.wait_send()
        amax = jnp.max(amax_buf[:, :])
        scale = amax / 448.0
        inv_scale = 448.0 / amax

        def stage_out(X, st_slot, src_slot, chunk_idx):
            if X.pending_store[st_slot] is not None:
                X.pending_store[st_slot].wait()
            X.st[st_slot] = (
                X.ag[src_slot].astype(jnp.float32) * scale
            ).astype(jnp.bfloat16)
            cp = pltpu.make_async_copy(
                X.st.at[st_slot],
                out_ref.at[pl.ds(chunk_idx * MC, MC), pl.ds(X.col0, NL)],
                X.store_sems.at[st_slot],
            )
            cp.start()
            X.pending_store[st_slot] = cp

        def ag_send(X, g):
            rd = pltpu.make_async_remote_copy(
                src_ref=X.ag.at[g % 3],
                dst_ref=X.ag.at[(g + 1) % 3],
                send_sem=X.ag_send.at[g % 3],
                recv_sem=X.ag_recv.at[(g + 1) % 3],
                device_id=(X.out_peer,),
                device_id_type=MESH,
            )
            rd.start()
            X.agsends.append(rd)

        for X in LANES:
            X.ag[0] = (X.wf32[0][:, :] * inv_scale).astype(jnp.float8_e4m3fn)
            ag_send(X, 0)
        for X in LANES:
            stage_out(X, 0, 0, X.own_row)

        for g in range(N_DEV - 1):
            for X in LANES:
                X.agsends[g].wait_recv()
                if g <= 13:
                    pltpu.semaphore_signal(X.ag_cred, 1, device_id=(X.in_peer,),
                                           device_id_type=MESH)
                if g < N_DEV - 2:
                    pltpu.semaphore_wait(X.ag_cred, 1)
                    ag_send(X, g + 1)
                X.agsends[g].wait_send()
                stage_out(X, (g + 1) % 2, (g + 1) % 3, X.ag_arrival(g))

        for X in LANES:
            for cp in X.pending_store:
                if cp is not None:
                    cp.wait()

    out_shape = jax.ShapeDtypeStruct((M, N), jnp.bfloat16)
    return pl.pallas_call(
        body,
        out_shape=out_shape,
        in_specs=[
            pl.BlockSpec(memory_space=pltpu.VMEM),
            pl.BlockSpec(memory_space=pltpu.VMEM),
        ],
        out_specs=pl.BlockSpec(memory_space=pl.ANY),
        scratch_shapes=(
            _LANE_SCRATCH * N_LANE
            + [
                pltpu.VMEM((N_DEV, 128), jnp.float32),
                pltpu.SemaphoreType.DMA((N_DEV,)),
                pltpu.SemaphoreType.DMA((N_DEV,)),
            ]
        ),
        compiler_params=pltpu.CompilerParams(
            collective_id=0, vmem_limit_bytes=100 * 1024 * 1024
        ),
    )(x16, w16)
